# baseline (speedup 1.0000x reference)
import functools

import jax
import jax.numpy as jnp
from jax import lax
from jax.experimental import pallas as pl
from jax.experimental.pallas import tpu as pltpu
from jax.experimental.pallas import tpu_sc as plsc

N = 10000
R = 10240
D = 2048
C = 128
NCH = D // C
E = 32000
EP = 32768
NWIN = 32
W = 64
STRIPE = R // 16

_mesh = plsc.VectorSubcoreMesh(core_axis_name="c", subcore_axis_name="s")


def _deg_body(dstw_hbm, ones_hbm, zeros_hbm, deg_hbm, dst_v, ones_v, deg_sh):
    ci = lax.axis_index("c")
    si = lax.axis_index("s")

    @pl.when(ci == 0)
    def _():
        pltpu.sync_copy(dstw_hbm.at[si], dst_v)
        pltpu.sync_copy(ones_hbm, ones_v)
        pltpu.sync_copy(zeros_hbm, deg_sh.at[pl.ds(si * STRIPE, STRIPE)])
        plsc.subcore_barrier()

        @pl.loop(0, NWIN)
        def _(j):
            pltpu.sync_copy(ones_v, deg_sh.at[dst_v.at[j]], add=True)

        plsc.subcore_barrier()
        pltpu.sync_copy(deg_sh.at[pl.ds(si * STRIPE, STRIPE)],
                        deg_hbm.at[pl.ds(si * STRIPE, STRIPE)])


def _sc_degree(dstw, ones_chunk, zeros_chunk):
    kern = pl.kernel(
        _deg_body,
        out_type=jax.ShapeDtypeStruct((R, C), jnp.float32),
        mesh=_mesh,
        scratch_types=[
            pltpu.VMEM((NWIN, W), jnp.int32),
            pltpu.VMEM((W, C), jnp.float32),
            pltpu.VMEM_SHARED((R, C), jnp.float32),
        ],
    )
    return kern(dstw, ones_chunk, zeros_chunk)


NBUF = 4


def _agg_body(h3_hbm, srcw_hbm, dstw_hbm, zeros_hbm, agg3_hbm,
              src_v, dst_v, rows_v, gsem, ssem, acc_sh):
    ci = lax.axis_index("c")
    si = lax.axis_index("s")
    pltpu.sync_copy(srcw_hbm.at[si], src_v)
    pltpu.sync_copy(dstw_hbm.at[si], dst_v)

    for half in range(2):
        @pl.when(ci == half)
        def _():
            for k in range(NCH // 2):
                c = half * (NCH // 2) + k
                pltpu.sync_copy(zeros_hbm,
                                acc_sh.at[pl.ds(si * STRIPE, STRIPE)])
                plsc.subcore_barrier()

                for b in range(NBUF):
                    pltpu.async_copy(h3_hbm.at[c].at[src_v.at[b]],
                                     rows_v.at[b], gsem.at[b])

                @pl.loop(0, NWIN, step=NBUF)
                def _(j0):
                    for b in range(NBUF):
                        j = j0 + b
                        pltpu.make_async_copy(
                            h3_hbm.at[c].at[src_v.at[j]], rows_v.at[b],
                            gsem.at[b]).wait()
                        pltpu.async_copy(rows_v.at[b],
                                         acc_sh.at[dst_v.at[j]],
                                         ssem.at[b], add=True)
                        pltpu.make_async_copy(
                            rows_v.at[b], acc_sh.at[dst_v.at[j]],
                            ssem.at[b]).wait()

                        @pl.when(j0 + NBUF < NWIN)
                        def _():
                            pltpu.async_copy(
                                h3_hbm.at[c].at[src_v.at[j + NBUF]],
                                rows_v.at[b], gsem.at[b])

                plsc.subcore_barrier()
                pltpu.sync_copy(acc_sh.at[pl.ds(si * STRIPE, STRIPE)],
                                agg3_hbm.at[c].at[pl.ds(si * STRIPE, STRIPE)])
                plsc.subcore_barrier()


def _sc_aggregate(h3, srcw, dstw, zeros_chunk):
    kern = pl.kernel(
        _agg_body,
        out_type=jax.ShapeDtypeStruct((NCH, R, C), jnp.float32),
        mesh=_mesh,
        scratch_types=[
            pltpu.VMEM((NWIN, W), jnp.int32),
            pltpu.VMEM((NWIN, W), jnp.int32),
            pltpu.VMEM((NBUF, W, C), jnp.float32),
            pltpu.SemaphoreType.DMA((NBUF,)),
            pltpu.SemaphoreType.DMA((NBUF,)),
            pltpu.VMEM_SHARED((R, C), jnp.float32),
        ],
    )
    return kern(h3, srcw, dstw, zeros_chunk)


BR = 1024
BC = 512


def _p1_body(h_ref, w_ref, b_ref, o_ref):
    a = h_ref[...].astype(jnp.bfloat16)
    acc = lax.dot_general(a, w_ref[...], (((1,), (1,)), ((), ())),
                          preferred_element_type=jnp.float32)
    o_ref[...] = acc + b_ref[...]


def _tc_p1(h, w_bf16, b_row):
    return pl.pallas_call(
        _p1_body,
        grid=(R // BR, D // BC),
        in_specs=[
            pl.BlockSpec((BR, D), lambda i, j: (i, 0)),
            pl.BlockSpec((BC, D), lambda i, j: (j, 0)),
            pl.BlockSpec((1, BC), lambda i, j: (0, j)),
        ],
        out_specs=pl.BlockSpec((BR, BC), lambda i, j: (i, j)),
        out_shape=jax.ShapeDtypeStruct((R, D), jnp.float32),
    )(h, w_bf16, b_row)


def _p2_body(tmp_ref, agg_ref, deg_ref, w_ref, o_ref, *, relu):
    scale = 1.0 / jnp.maximum(deg_ref[...], 1.0)
    a = (agg_ref[...] * scale).astype(jnp.bfloat16)
    acc = lax.dot_general(a, w_ref[...], (((1,), (1,)), ((), ())),
                          preferred_element_type=jnp.float32)
    out = tmp_ref[...] + acc
    if relu:
        out = jnp.maximum(out, 0.0)
    o_ref[...] = out


def _tc_p2(tmp, agg, deg_col, w_bf16, relu):
    return pl.pallas_call(
        functools.partial(_p2_body, relu=relu),
        grid=(R // BR, D // BC),
        in_specs=[
            pl.BlockSpec((BR, BC), lambda i, j: (i, j)),
            pl.BlockSpec((BR, D), lambda i, j: (i, 0)),
            pl.BlockSpec((BR, 1), lambda i, j: (i, 0)),
            pl.BlockSpec((BC, D), lambda i, j: (j, 0)),
        ],
        out_specs=pl.BlockSpec((BR, BC), lambda i, j: (i, j)),
        out_shape=jax.ShapeDtypeStruct((R, D), jnp.float32),
    )(tmp, agg, deg_col, w_bf16)


def _chunk_body(x_ref, o_ref):
    o_ref[...] = x_ref[...][None]


def _tc_chunk(h):
    return pl.pallas_call(
        _chunk_body,
        grid=(NCH, R // BR),
        in_specs=[pl.BlockSpec((BR, C), lambda c, i: (i, c))],
        out_specs=pl.BlockSpec((1, BR, C), lambda c, i: (c, i, 0)),
        out_shape=jax.ShapeDtypeStruct((NCH, R, C), jnp.float32),
    )(h)


def _unchunk_body(x_ref, o_ref):
    o_ref[...] = x_ref[...][0]


def _tc_unchunk(a3):
    return pl.pallas_call(
        _unchunk_body,
        grid=(NCH, R // BR),
        in_specs=[pl.BlockSpec((1, BR, C), lambda c, i: (c, i, 0))],
        out_specs=pl.BlockSpec((BR, C), lambda c, i: (i, c)),
        out_shape=jax.ShapeDtypeStruct((R, D), jnp.float32),
    )(a3)


def kernel(x, edge_index, W_l_0, W_r_0, b_0, W_l_1, W_r_1, b_1,
           W_l_2, W_r_2, b_2, W_l_3, W_r_3, b_3):
    src = edge_index[0]
    dst = edge_index[1]
    npad = EP - E
    pad_src = jnp.zeros((npad,), jnp.int32)
    pad_dst = N + (jnp.arange(npad, dtype=jnp.int32) % (R - N))
    srcw = jnp.concatenate([src, pad_src]).reshape(16, NWIN, W)
    dstw = jnp.concatenate([dst, pad_dst]).reshape(16, NWIN, W)

    zeros_chunk = jnp.zeros((STRIPE, C), jnp.float32)
    ones_chunk = jnp.ones((W, C), jnp.float32)

    Wl = [W_l_0, W_l_1, W_l_2, W_l_3]
    Wr = [W_r_0, W_r_1, W_r_2, W_r_3]
    bs = [b_0, b_1, b_2, b_3]
    Wlb = [w.astype(jnp.bfloat16) for w in Wl]
    Wrb = [w.astype(jnp.bfloat16) for w in Wr]
    brow = [b.reshape(1, D) for b in bs]

    degw = _sc_degree(dstw, ones_chunk, zeros_chunk)
    deg_col = degw[:, 0:1]

    h = jnp.pad(x, ((0, R - N), (0, 0)))
    h3 = _tc_chunk(h)
    for l in range(4):
        agg3 = _sc_aggregate(h3, srcw, dstw, zeros_chunk)
        tmp = _tc_p1(h, Wrb[l], brow[l])
        agg = _tc_unchunk(agg3)
        h = _tc_p2(tmp, agg, deg_col, Wlb[l], relu=(l % 2 == 0))
        if l < 3:
            h3 = _tc_chunk(h)
    return h[:N]


# speedup vs baseline: 1.1545x; 1.1545x over previous
import functools

import jax
import jax.numpy as jnp
from jax import lax
from jax.experimental import pallas as pl
from jax.experimental.pallas import tpu as pltpu
from jax.experimental.pallas import tpu_sc as plsc

N = 10000
R = 10240
D = 2048
C = 128
NCH = D // C
E = 32000
EP = 32768
NWIN = 32
W = 64
STRIPE = R // 16

_mesh = plsc.VectorSubcoreMesh(core_axis_name="c", subcore_axis_name="s")


def _deg_body(dstw_hbm, ones_hbm, zeros_hbm, deg_hbm, dst_v, ones_v, deg_sh):
    ci = lax.axis_index("c")
    si = lax.axis_index("s")

    @pl.when(ci == 0)
    def _():
        pltpu.sync_copy(dstw_hbm.at[si], dst_v)
        pltpu.sync_copy(ones_hbm, ones_v)
        pltpu.sync_copy(zeros_hbm, deg_sh.at[pl.ds(si * STRIPE, STRIPE)])
        plsc.subcore_barrier()

        @pl.loop(0, NWIN)
        def _(j):
            pltpu.sync_copy(ones_v, deg_sh.at[dst_v.at[j]], add=True)

        plsc.subcore_barrier()
        pltpu.sync_copy(deg_sh.at[pl.ds(si * STRIPE, STRIPE)],
                        deg_hbm.at[pl.ds(si * STRIPE, STRIPE)])


def _sc_degree(dstw, ones_chunk, zeros_chunk):
    kern = pl.kernel(
        _deg_body,
        out_type=jax.ShapeDtypeStruct((R, C), jnp.float32),
        mesh=_mesh,
        scratch_types=[
            pltpu.VMEM((NWIN, W), jnp.int32),
            pltpu.VMEM((W, C), jnp.float32),
            pltpu.VMEM_SHARED((R, C), jnp.float32),
        ],
    )
    return kern(dstw, ones_chunk, zeros_chunk)


NBUF = 4


def _agg_body(h3_hbm, srcw_hbm, dstw_hbm, zeros_hbm, agg3_hbm,
              src_v, dst_v, rows_v, gsem, ssem, acc_sh):
    ci = lax.axis_index("c")
    si = lax.axis_index("s")
    pltpu.sync_copy(srcw_hbm.at[si], src_v)
    pltpu.sync_copy(dstw_hbm.at[si], dst_v)

    for half in range(2):
        @pl.when(ci == half)
        def _():
            for k in range(NCH // 2):
                c = half * (NCH // 2) + k
                pltpu.sync_copy(zeros_hbm,
                                acc_sh.at[pl.ds(si * STRIPE, STRIPE)])
                plsc.subcore_barrier()

                for b in range(NBUF):
                    pltpu.async_copy(h3_hbm.at[c].at[src_v.at[b]],
                                     rows_v.at[b], gsem.at[b])

                @pl.loop(0, NWIN, step=NBUF)
                def _(j0):
                    for b in range(NBUF):
                        j = j0 + b
                        pltpu.make_async_copy(
                            h3_hbm.at[c].at[src_v.at[j]], rows_v.at[b],
                            gsem.at[b]).wait()
                        pltpu.async_copy(rows_v.at[b],
                                         acc_sh.at[dst_v.at[j]],
                                         ssem.at[b], add=True)
                        pltpu.make_async_copy(
                            rows_v.at[b], acc_sh.at[dst_v.at[j]],
                            ssem.at[b]).wait()

                        @pl.when(j0 + NBUF < NWIN)
                        def _():
                            pltpu.async_copy(
                                h3_hbm.at[c].at[src_v.at[j + NBUF]],
                                rows_v.at[b], gsem.at[b])

                plsc.subcore_barrier()
                pltpu.sync_copy(acc_sh.at[pl.ds(si * STRIPE, STRIPE)],
                                agg3_hbm.at[c].at[pl.ds(si * STRIPE, STRIPE)])
                plsc.subcore_barrier()


def _sc_aggregate(h3, srcw, dstw, zeros_chunk):
    kern = pl.kernel(
        _agg_body,
        out_type=jax.ShapeDtypeStruct((NCH, R, C), jnp.float32),
        mesh=_mesh,
        scratch_types=[
            pltpu.VMEM((NWIN, W), jnp.int32),
            pltpu.VMEM((NWIN, W), jnp.int32),
            pltpu.VMEM((NBUF, W, C), jnp.float32),
            pltpu.SemaphoreType.DMA((NBUF,)),
            pltpu.SemaphoreType.DMA((NBUF,)),
            pltpu.VMEM_SHARED((R, C), jnp.float32),
        ],
    )
    return kern(h3, srcw, dstw, zeros_chunk)


BR = 1024
BC = 512


KC = 256
NKC = D // KC


def _p1_body(a3_ref, w_ref, b_ref, o_ref):
    k = pl.program_id(1)
    a3 = a3_ref[...]
    a = jnp.concatenate([a3[cc] for cc in range(KC // C)], axis=1)
    part = lax.dot_general(a.astype(jnp.bfloat16), w_ref[...],
                           (((1,), (1,)), ((), ())),
                           preferred_element_type=jnp.float32)

    @pl.when(k == 0)
    def _():
        o_ref[...] = part + b_ref[...]

    @pl.when(k > 0)
    def _():
        o_ref[...] += part


def _tc_p1(h3, w_bf16, b_row):
    return pl.pallas_call(
        _p1_body,
        grid=(R // BR, NKC),
        in_specs=[
            pl.BlockSpec((KC // C, BR, C), lambda i, k: (k, i, 0)),
            pl.BlockSpec((D, KC), lambda i, k: (0, k)),
            pl.BlockSpec((1, D), lambda i, k: (0, 0)),
        ],
        out_specs=pl.BlockSpec((BR, D), lambda i, k: (i, 0)),
        out_shape=jax.ShapeDtypeStruct((R, D), jnp.float32),
    )(h3, w_bf16, b_row)


def _p2_body(tmp_ref, a3_ref, deg_ref, w_ref, o_ref, acc_ref, *, relu,
             emit_std):
    k = pl.program_id(1)
    scale = 1.0 / jnp.maximum(deg_ref[...], 1.0)
    a3 = a3_ref[...]
    a = jnp.concatenate([a3[cc] for cc in range(KC // C)], axis=1)
    a = (a * scale).astype(jnp.bfloat16)
    part = lax.dot_general(a, w_ref[...], (((1,), (1,)), ((), ())),
                           preferred_element_type=jnp.float32)

    @pl.when(k == 0)
    def _():
        acc_ref[...] = tmp_ref[...] + part

    @pl.when(k > 0)
    def _():
        acc_ref[...] += part

    @pl.when(k == NKC - 1)
    def _():
        out = acc_ref[...]
        if relu:
            out = jnp.maximum(out, 0.0)
        if emit_std:
            o_ref[...] = out
        else:
            for cc in range(NCH):
                o_ref[cc] = out[:, cc * C:(cc + 1) * C]


def _tc_p2(tmp, agg3, deg_col, w_bf16, relu, emit_std):
    if emit_std:
        out_spec = pl.BlockSpec((BR, D), lambda i, k: (i, 0))
        out_shape = jax.ShapeDtypeStruct((R, D), jnp.float32)
    else:
        out_spec = pl.BlockSpec((NCH, BR, C), lambda i, k: (0, i, 0))
        out_shape = jax.ShapeDtypeStruct((NCH, R, C), jnp.float32)
    return pl.pallas_call(
        functools.partial(_p2_body, relu=relu, emit_std=emit_std),
        grid=(R // BR, NKC),
        in_specs=[
            pl.BlockSpec((BR, D), lambda i, k: (i, 0)),
            pl.BlockSpec((KC // C, BR, C), lambda i, k: (k, i, 0)),
            pl.BlockSpec((BR, 1), lambda i, k: (i, 0)),
            pl.BlockSpec((D, KC), lambda i, k: (0, k)),
        ],
        out_specs=out_spec,
        out_shape=out_shape,
        scratch_shapes=[pltpu.VMEM((BR, D), jnp.float32)],
    )(tmp, agg3, deg_col, w_bf16)


def _chunk_body(x_ref, o_ref):
    o_ref[...] = x_ref[...][None]


def _tc_chunk(h):
    return pl.pallas_call(
        _chunk_body,
        grid=(NCH, R // BR),
        in_specs=[pl.BlockSpec((BR, C), lambda c, i: (i, c))],
        out_specs=pl.BlockSpec((1, BR, C), lambda c, i: (c, i, 0)),
        out_shape=jax.ShapeDtypeStruct((NCH, R, C), jnp.float32),
    )(h)


def kernel(x, edge_index, W_l_0, W_r_0, b_0, W_l_1, W_r_1, b_1,
           W_l_2, W_r_2, b_2, W_l_3, W_r_3, b_3):
    src = edge_index[0]
    dst = edge_index[1]
    npad = EP - E
    pad_src = jnp.zeros((npad,), jnp.int32)
    pad_dst = N + (jnp.arange(npad, dtype=jnp.int32) % (R - N))
    srcw = jnp.concatenate([src, pad_src]).reshape(16, NWIN, W)
    dstw = jnp.concatenate([dst, pad_dst]).reshape(16, NWIN, W)

    zeros_chunk = jnp.zeros((STRIPE, C), jnp.float32)
    ones_chunk = jnp.ones((W, C), jnp.float32)

    Wl = [W_l_0, W_l_1, W_l_2, W_l_3]
    Wr = [W_r_0, W_r_1, W_r_2, W_r_3]
    bs = [b_0, b_1, b_2, b_3]
    Wlb = [w.astype(jnp.bfloat16) for w in Wl]
    Wrb = [w.astype(jnp.bfloat16) for w in Wr]
    brow = [b.reshape(1, D) for b in bs]

    degw = _sc_degree(dstw, ones_chunk, zeros_chunk)
    deg_col = degw[:, 0:1]

    h3 = _tc_chunk(jnp.pad(x, ((0, R - N), (0, 0))))
    for l in range(4):
        agg3 = _sc_aggregate(h3, srcw, dstw, zeros_chunk)
        tmp = _tc_p1(h3, Wrb[l], brow[l])
        out = _tc_p2(tmp, agg3, deg_col, Wlb[l],
                     relu=(l % 2 == 0), emit_std=(l == 3))
        h3 = out
    return out[:N]


# speedup vs baseline: 1.1592x; 1.0040x over previous
import functools

import jax
import jax.numpy as jnp
from jax import lax
from jax.experimental import pallas as pl
from jax.experimental.pallas import tpu as pltpu
from jax.experimental.pallas import tpu_sc as plsc

N = 10000
R = 10240
D = 2048
C = 128
NCH = D // C
E = 32000
EP = 32768
NWIN = 32
W = 64
STRIPE = R // 16

_mesh = plsc.VectorSubcoreMesh(core_axis_name="c", subcore_axis_name="s")


def _deg_body(dstw_hbm, ones_hbm, zeros_hbm, deg_hbm, dst_v, ones_v, deg_sh):
    ci = lax.axis_index("c")
    si = lax.axis_index("s")

    @pl.when(ci == 0)
    def _():
        pltpu.sync_copy(dstw_hbm.at[si], dst_v)
        pltpu.sync_copy(ones_hbm, ones_v)
        pltpu.sync_copy(zeros_hbm, deg_sh.at[pl.ds(si * STRIPE, STRIPE)])
        plsc.subcore_barrier()

        @pl.loop(0, NWIN)
        def _(j):
            pltpu.sync_copy(ones_v, deg_sh.at[dst_v.at[j]], add=True)

        plsc.subcore_barrier()
        pltpu.sync_copy(deg_sh.at[pl.ds(si * STRIPE, STRIPE)],
                        deg_hbm.at[pl.ds(si * STRIPE, STRIPE)])


def _sc_degree(dstw, ones_chunk, zeros_chunk):
    kern = pl.kernel(
        _deg_body,
        out_type=jax.ShapeDtypeStruct((R, C), jnp.float32),
        mesh=_mesh,
        scratch_types=[
            pltpu.VMEM((NWIN, W), jnp.int32),
            pltpu.VMEM((W, C), jnp.float32),
            pltpu.VMEM_SHARED((R, C), jnp.float32),
        ],
    )
    return kern(dstw, ones_chunk, zeros_chunk)


NBUF = 4


def _agg_body(h3_hbm, srcw_hbm, dstw_hbm, zeros_hbm, agg3_hbm,
              src_v, dst_v, rows_v, gsem, ssem, acc_sh):
    ci = lax.axis_index("c")
    si = lax.axis_index("s")
    pltpu.sync_copy(srcw_hbm.at[si], src_v)
    pltpu.sync_copy(dstw_hbm.at[si], dst_v)

    for half in range(2):
        @pl.when(ci == half)
        def _():
            base = half * (NCH // 2)
            for b in range(NBUF):
                pltpu.async_copy(h3_hbm.at[base].at[src_v.at[b]],
                                 rows_v.at[b], gsem.at[b])
            for k in range(NCH // 2):
                c = base + k
                pltpu.sync_copy(zeros_hbm,
                                acc_sh.at[pl.ds(si * STRIPE, STRIPE)])
                plsc.subcore_barrier()

                @pl.loop(0, NWIN, step=NBUF)
                def _(j0):
                    for b in range(NBUF):
                        j = j0 + b
                        pltpu.make_async_copy(
                            h3_hbm.at[c].at[src_v.at[j]], rows_v.at[b],
                            gsem.at[b]).wait()
                        pltpu.async_copy(rows_v.at[b],
                                         acc_sh.at[dst_v.at[j]],
                                         ssem.at[b], add=True)
                        pltpu.make_async_copy(
                            rows_v.at[b], acc_sh.at[dst_v.at[j]],
                            ssem.at[b]).wait()

                        @pl.when(j0 + NBUF < NWIN)
                        def _():
                            pltpu.async_copy(
                                h3_hbm.at[c].at[src_v.at[j + NBUF]],
                                rows_v.at[b], gsem.at[b])

                if k + 1 < NCH // 2:
                    for b in range(NBUF):
                        pltpu.async_copy(h3_hbm.at[c + 1].at[src_v.at[b]],
                                         rows_v.at[b], gsem.at[b])
                plsc.subcore_barrier()
                pltpu.sync_copy(acc_sh.at[pl.ds(si * STRIPE, STRIPE)],
                                agg3_hbm.at[c].at[pl.ds(si * STRIPE, STRIPE)])
                plsc.subcore_barrier()


def _sc_aggregate(h3, srcw, dstw, zeros_chunk):
    kern = pl.kernel(
        _agg_body,
        out_type=jax.ShapeDtypeStruct((NCH, R, C), jnp.float32),
        mesh=_mesh,
        scratch_types=[
            pltpu.VMEM((NWIN, W), jnp.int32),
            pltpu.VMEM((NWIN, W), jnp.int32),
            pltpu.VMEM((NBUF, W, C), jnp.float32),
            pltpu.SemaphoreType.DMA((NBUF,)),
            pltpu.SemaphoreType.DMA((NBUF,)),
            pltpu.VMEM_SHARED((R, C), jnp.float32),
        ],
    )
    return kern(h3, srcw, dstw, zeros_chunk)


BR = 1024
BC = 512


KC = 256
NKC = D // KC


def _p1_body(a3_ref, w_ref, b_ref, o_ref):
    k = pl.program_id(1)
    a3 = a3_ref[...]
    a = jnp.concatenate([a3[cc] for cc in range(KC // C)], axis=1)
    part = lax.dot_general(a.astype(jnp.bfloat16), w_ref[...],
                           (((1,), (1,)), ((), ())),
                           preferred_element_type=jnp.float32)

    @pl.when(k == 0)
    def _():
        o_ref[...] = part + b_ref[...]

    @pl.when(k > 0)
    def _():
        o_ref[...] += part


def _tc_p1(h3, w_bf16, b_row):
    return pl.pallas_call(
        _p1_body,
        grid=(R // BR, NKC),
        in_specs=[
            pl.BlockSpec((KC // C, BR, C), lambda i, k: (k, i, 0)),
            pl.BlockSpec((D, KC), lambda i, k: (0, k)),
            pl.BlockSpec((1, D), lambda i, k: (0, 0)),
        ],
        out_specs=pl.BlockSpec((BR, D), lambda i, k: (i, 0)),
        out_shape=jax.ShapeDtypeStruct((R, D), jnp.float32),
    )(h3, w_bf16, b_row)


def _p2_body(tmp_ref, a3_ref, deg_ref, w_ref, o_ref, acc_ref, *, relu,
             emit_std):
    k = pl.program_id(1)
    scale = 1.0 / jnp.maximum(deg_ref[...], 1.0)
    a3 = a3_ref[...]
    a = jnp.concatenate([a3[cc] for cc in range(KC // C)], axis=1)
    a = (a * scale).astype(jnp.bfloat16)
    part = lax.dot_general(a, w_ref[...], (((1,), (1,)), ((), ())),
                           preferred_element_type=jnp.float32)

    @pl.when(k == 0)
    def _():
        acc_ref[...] = tmp_ref[...] + part

    @pl.when(k > 0)
    def _():
        acc_ref[...] += part

    @pl.when(k == NKC - 1)
    def _():
        out = acc_ref[...]
        if relu:
            out = jnp.maximum(out, 0.0)
        if emit_std:
            o_ref[...] = out
        else:
            for cc in range(NCH):
                o_ref[cc] = out[:, cc * C:(cc + 1) * C]


def _tc_p2(tmp, agg3, deg_col, w_bf16, relu, emit_std):
    if emit_std:
        out_spec = pl.BlockSpec((BR, D), lambda i, k: (i, 0))
        out_shape = jax.ShapeDtypeStruct((R, D), jnp.float32)
    else:
        out_spec = pl.BlockSpec((NCH, BR, C), lambda i, k: (0, i, 0))
        out_shape = jax.ShapeDtypeStruct((NCH, R, C), jnp.float32)
    return pl.pallas_call(
        functools.partial(_p2_body, relu=relu, emit_std=emit_std),
        grid=(R // BR, NKC),
        in_specs=[
            pl.BlockSpec((BR, D), lambda i, k: (i, 0)),
            pl.BlockSpec((KC // C, BR, C), lambda i, k: (k, i, 0)),
            pl.BlockSpec((BR, 1), lambda i, k: (i, 0)),
            pl.BlockSpec((D, KC), lambda i, k: (0, k)),
        ],
        out_specs=out_spec,
        out_shape=out_shape,
        scratch_shapes=[pltpu.VMEM((BR, D), jnp.float32)],
    )(tmp, agg3, deg_col, w_bf16)


def _chunk_body(x_ref, o_ref):
    o_ref[...] = x_ref[...][None]


def _tc_chunk(h):
    return pl.pallas_call(
        _chunk_body,
        grid=(NCH, R // BR),
        in_specs=[pl.BlockSpec((BR, C), lambda c, i: (i, c))],
        out_specs=pl.BlockSpec((1, BR, C), lambda c, i: (c, i, 0)),
        out_shape=jax.ShapeDtypeStruct((NCH, R, C), jnp.float32),
    )(h)


def kernel(x, edge_index, W_l_0, W_r_0, b_0, W_l_1, W_r_1, b_1,
           W_l_2, W_r_2, b_2, W_l_3, W_r_3, b_3):
    src = edge_index[0]
    dst = edge_index[1]
    npad = EP - E
    pad_src = jnp.zeros((npad,), jnp.int32)
    pad_dst = N + (jnp.arange(npad, dtype=jnp.int32) % (R - N))
    srcw = jnp.concatenate([src, pad_src]).reshape(16, NWIN, W)
    dstw = jnp.concatenate([dst, pad_dst]).reshape(16, NWIN, W)

    zeros_chunk = jnp.zeros((STRIPE, C), jnp.float32)
    ones_chunk = jnp.ones((W, C), jnp.float32)

    Wl = [W_l_0, W_l_1, W_l_2, W_l_3]
    Wr = [W_r_0, W_r_1, W_r_2, W_r_3]
    bs = [b_0, b_1, b_2, b_3]
    Wlb = [w.astype(jnp.bfloat16) for w in Wl]
    Wrb = [w.astype(jnp.bfloat16) for w in Wr]
    brow = [b.reshape(1, D) for b in bs]

    degw = _sc_degree(dstw, ones_chunk, zeros_chunk)
    deg_col = degw[:, 0:1]

    h3 = _tc_chunk(jnp.pad(x, ((0, R - N), (0, 0))))
    for l in range(4):
        agg3 = _sc_aggregate(h3, srcw, dstw, zeros_chunk)
        tmp = _tc_p1(h3, Wrb[l], brow[l])
        out = _tc_p2(tmp, agg3, deg_col, Wlb[l],
                     relu=(l % 2 == 0), emit_std=(l == 3))
        h3 = out
    return out[:N]


# speedup vs baseline: 1.2181x; 1.0509x over previous
import functools

import jax
import jax.numpy as jnp
from jax import lax
from jax.experimental import pallas as pl
from jax.experimental.pallas import tpu as pltpu
from jax.experimental.pallas import tpu_sc as plsc

N = 10000
R = 10240
D = 2048
C = 128
NCH = D // C
E = 32000
EP = 32768
NWIN = 32
W = 64
STRIPE = R // 16

_mesh = plsc.VectorSubcoreMesh(core_axis_name="c", subcore_axis_name="s")


def _deg_body(dstw_hbm, ones_hbm, zeros_hbm, deg_hbm, dst_v, ones_v, deg_sh):
    ci = lax.axis_index("c")
    si = lax.axis_index("s")

    @pl.when(ci == 0)
    def _():
        pltpu.sync_copy(dstw_hbm.at[si], dst_v)
        pltpu.sync_copy(ones_hbm, ones_v)
        pltpu.sync_copy(zeros_hbm, deg_sh.at[pl.ds(si * STRIPE, STRIPE)])
        plsc.subcore_barrier()

        @pl.loop(0, NWIN)
        def _(j):
            pltpu.sync_copy(ones_v, deg_sh.at[dst_v.at[j]], add=True)

        plsc.subcore_barrier()
        pltpu.sync_copy(deg_sh.at[pl.ds(si * STRIPE, STRIPE)],
                        deg_hbm.at[pl.ds(si * STRIPE, STRIPE)])


def _sc_degree(dstw, ones_chunk, zeros_chunk):
    kern = pl.kernel(
        _deg_body,
        out_type=jax.ShapeDtypeStruct((R, C), jnp.float32),
        mesh=_mesh,
        scratch_types=[
            pltpu.VMEM((NWIN, W), jnp.int32),
            pltpu.VMEM((W, C), jnp.float32),
            pltpu.VMEM_SHARED((R, C), jnp.float32),
        ],
    )
    return kern(dstw, ones_chunk, zeros_chunk)


NBUF = 4


def _agg_body(h3_hbm, srcw_hbm, dstw_hbm, zeros_hbm, agg3_hbm,
              src_v, dst_v, rows_v, zeros_v, gsem, ssem, zsem, acc_sh):
    ci = lax.axis_index("c")
    si = lax.axis_index("s")
    pltpu.sync_copy(srcw_hbm.at[si], src_v)
    pltpu.sync_copy(dstw_hbm.at[si], dst_v)
    pltpu.sync_copy(zeros_hbm, zeros_v)

    for half in range(2):
        @pl.when(ci == half)
        def _():
            base = half * (NCH // 2)
            for b in range(NBUF):
                pltpu.async_copy(h3_hbm.at[base].at[src_v.at[b]],
                                 rows_v.at[b], gsem.at[b])
            for k in range(NCH // 2):
                c = base + k

                @pl.loop(0, STRIPE, step=64)
                def _(z):
                    pltpu.async_copy(
                        zeros_v, acc_sh.at[pl.ds(si * STRIPE + z, 64)],
                        zsem)

                @pl.loop(0, STRIPE, step=64)
                def _(z):
                    pltpu.make_async_copy(
                        zeros_v, acc_sh.at[pl.ds(si * STRIPE + z, 64)],
                        zsem).wait()

                plsc.subcore_barrier()

                @pl.loop(0, NWIN, step=NBUF)
                def _(j0):
                    for b in range(NBUF):
                        j = j0 + b
                        pltpu.make_async_copy(
                            h3_hbm.at[c].at[src_v.at[j]], rows_v.at[b],
                            gsem.at[b]).wait()
                        pltpu.async_copy(rows_v.at[b],
                                         acc_sh.at[dst_v.at[j]],
                                         ssem.at[b], add=True)
                        pltpu.make_async_copy(
                            rows_v.at[b], acc_sh.at[dst_v.at[j]],
                            ssem.at[b]).wait()

                        @pl.when(j0 + NBUF < NWIN)
                        def _():
                            pltpu.async_copy(
                                h3_hbm.at[c].at[src_v.at[j + NBUF]],
                                rows_v.at[b], gsem.at[b])

                if k + 1 < NCH // 2:
                    for b in range(NBUF):
                        pltpu.async_copy(h3_hbm.at[c + 1].at[src_v.at[b]],
                                         rows_v.at[b], gsem.at[b])
                plsc.subcore_barrier()
                pltpu.sync_copy(acc_sh.at[pl.ds(si * STRIPE, STRIPE)],
                                agg3_hbm.at[c].at[pl.ds(si * STRIPE, STRIPE)])
                plsc.subcore_barrier()


def _sc_aggregate(h3, srcw, dstw, zeros_chunk):
    kern = pl.kernel(
        _agg_body,
        out_type=jax.ShapeDtypeStruct((NCH, R, C), jnp.float32),
        mesh=_mesh,
        scratch_types=[
            pltpu.VMEM((NWIN, W), jnp.int32),
            pltpu.VMEM((NWIN, W), jnp.int32),
            pltpu.VMEM((NBUF, W, C), jnp.float32),
            pltpu.VMEM((64, C), jnp.float32),
            pltpu.SemaphoreType.DMA((NBUF,)),
            pltpu.SemaphoreType.DMA((NBUF,)),
            pltpu.SemaphoreType.DMA,
            pltpu.VMEM_SHARED((R, C), jnp.float32),
        ],
    )
    return kern(h3, srcw, dstw, zeros_chunk)


BR = 1024
BC = 512


KC = 256
NKC = D // KC


def _p1_body(a3_ref, w_ref, b_ref, o_ref):
    k = pl.program_id(1)
    a3 = a3_ref[...]
    a = jnp.concatenate([a3[cc] for cc in range(KC // C)], axis=1)
    part = lax.dot_general(a.astype(jnp.bfloat16), w_ref[...],
                           (((1,), (1,)), ((), ())),
                           preferred_element_type=jnp.float32)

    @pl.when(k == 0)
    def _():
        o_ref[...] = part + b_ref[...]

    @pl.when(k > 0)
    def _():
        o_ref[...] += part


def _tc_p1(h3, w_bf16, b_row):
    return pl.pallas_call(
        _p1_body,
        grid=(R // BR, NKC),
        in_specs=[
            pl.BlockSpec((KC // C, BR, C), lambda i, k: (k, i, 0)),
            pl.BlockSpec((D, KC), lambda i, k: (0, k)),
            pl.BlockSpec((1, D), lambda i, k: (0, 0)),
        ],
        out_specs=pl.BlockSpec((BR, D), lambda i, k: (i, 0)),
        out_shape=jax.ShapeDtypeStruct((R, D), jnp.float32),
    )(h3, w_bf16, b_row)


def _p2_body(tmp_ref, a3_ref, deg_ref, w_ref, o_ref, acc_ref, *, relu,
             emit_std):
    k = pl.program_id(1)
    scale = 1.0 / jnp.maximum(deg_ref[...], 1.0)
    a3 = a3_ref[...]
    a = jnp.concatenate([a3[cc] for cc in range(KC // C)], axis=1)
    a = (a * scale).astype(jnp.bfloat16)
    part = lax.dot_general(a, w_ref[...], (((1,), (1,)), ((), ())),
                           preferred_element_type=jnp.float32)

    @pl.when(k == 0)
    def _():
        acc_ref[...] = tmp_ref[...] + part

    @pl.when(k > 0)
    def _():
        acc_ref[...] += part

    @pl.when(k == NKC - 1)
    def _():
        out = acc_ref[...]
        if relu:
            out = jnp.maximum(out, 0.0)
        if emit_std:
            o_ref[...] = out
        else:
            for cc in range(NCH):
                o_ref[cc] = out[:, cc * C:(cc + 1) * C]


def _tc_p2(tmp, agg3, deg_col, w_bf16, relu, emit_std):
    if emit_std:
        out_spec = pl.BlockSpec((BR, D), lambda i, k: (i, 0))
        out_shape = jax.ShapeDtypeStruct((R, D), jnp.float32)
    else:
        out_spec = pl.BlockSpec((NCH, BR, C), lambda i, k: (0, i, 0))
        out_shape = jax.ShapeDtypeStruct((NCH, R, C), jnp.float32)
    return pl.pallas_call(
        functools.partial(_p2_body, relu=relu, emit_std=emit_std),
        grid=(R // BR, NKC),
        in_specs=[
            pl.BlockSpec((BR, D), lambda i, k: (i, 0)),
            pl.BlockSpec((KC // C, BR, C), lambda i, k: (k, i, 0)),
            pl.BlockSpec((BR, 1), lambda i, k: (i, 0)),
            pl.BlockSpec((D, KC), lambda i, k: (0, k)),
        ],
        out_specs=out_spec,
        out_shape=out_shape,
        scratch_shapes=[pltpu.VMEM((BR, D), jnp.float32)],
    )(tmp, agg3, deg_col, w_bf16)


def _chunk_body(x_ref, o_ref):
    o_ref[...] = x_ref[...][None]


def _tc_chunk(h):
    return pl.pallas_call(
        _chunk_body,
        grid=(NCH, R // BR),
        in_specs=[pl.BlockSpec((BR, C), lambda c, i: (i, c))],
        out_specs=pl.BlockSpec((1, BR, C), lambda c, i: (c, i, 0)),
        out_shape=jax.ShapeDtypeStruct((NCH, R, C), jnp.float32),
    )(h)


def kernel(x, edge_index, W_l_0, W_r_0, b_0, W_l_1, W_r_1, b_1,
           W_l_2, W_r_2, b_2, W_l_3, W_r_3, b_3):
    src = edge_index[0]
    dst = edge_index[1]
    npad = EP - E
    pad_src = jnp.zeros((npad,), jnp.int32)
    pad_dst = N + (jnp.arange(npad, dtype=jnp.int32) % (R - N))
    srcw = jnp.concatenate([src, pad_src]).reshape(16, NWIN, W)
    dstw = jnp.concatenate([dst, pad_dst]).reshape(16, NWIN, W)

    zeros_stripe = jnp.zeros((STRIPE, C), jnp.float32)
    zeros64 = jnp.zeros((64, C), jnp.float32)
    ones_chunk = jnp.ones((W, C), jnp.float32)

    Wl = [W_l_0, W_l_1, W_l_2, W_l_3]
    Wr = [W_r_0, W_r_1, W_r_2, W_r_3]
    bs = [b_0, b_1, b_2, b_3]
    Wlb = [w.astype(jnp.bfloat16) for w in Wl]
    Wrb = [w.astype(jnp.bfloat16) for w in Wr]
    brow = [b.reshape(1, D) for b in bs]

    degw = _sc_degree(dstw, ones_chunk, zeros_stripe)
    deg_col = degw[:, 0:1]

    h3 = _tc_chunk(jnp.pad(x, ((0, R - N), (0, 0))))
    for l in range(4):
        agg3 = _sc_aggregate(h3, srcw, dstw, zeros64)
        tmp = _tc_p1(h3, Wrb[l], brow[l])
        out = _tc_p2(tmp, agg3, deg_col, Wlb[l],
                     relu=(l % 2 == 0), emit_std=(l == 3))
        h3 = out
    return out[:N]


# speedup vs baseline: 1.2185x; 1.0003x over previous
import functools

import jax
import jax.numpy as jnp
from jax import lax
from jax.experimental import pallas as pl
from jax.experimental.pallas import tpu as pltpu
from jax.experimental.pallas import tpu_sc as plsc

N = 10000
R = 10240
D = 2048
C = 128
NCH = D // C
E = 32000
EP = 32768
NWIN = 16
W = 128
STRIPE = R // 16

_mesh = plsc.VectorSubcoreMesh(core_axis_name="c", subcore_axis_name="s")


def _deg_body(dstw_hbm, ones_hbm, zeros_hbm, deg_hbm, dst_v, ones_v, deg_sh):
    ci = lax.axis_index("c")
    si = lax.axis_index("s")

    @pl.when(ci == 0)
    def _():
        pltpu.sync_copy(dstw_hbm.at[si], dst_v)
        pltpu.sync_copy(ones_hbm, ones_v)
        pltpu.sync_copy(zeros_hbm, deg_sh.at[pl.ds(si * STRIPE, STRIPE)])
        plsc.subcore_barrier()

        @pl.loop(0, NWIN)
        def _(j):
            pltpu.sync_copy(ones_v, deg_sh.at[dst_v.at[j]], add=True)

        plsc.subcore_barrier()
        pltpu.sync_copy(deg_sh.at[pl.ds(si * STRIPE, STRIPE)],
                        deg_hbm.at[pl.ds(si * STRIPE, STRIPE)])


def _sc_degree(dstw, ones_chunk, zeros_chunk):
    kern = pl.kernel(
        _deg_body,
        out_type=jax.ShapeDtypeStruct((R, C), jnp.float32),
        mesh=_mesh,
        scratch_types=[
            pltpu.VMEM((NWIN, W), jnp.int32),
            pltpu.VMEM((W, C), jnp.float32),
            pltpu.VMEM_SHARED((R, C), jnp.float32),
        ],
    )
    return kern(dstw, ones_chunk, zeros_chunk)


NBUF = 2


def _agg_body(h3_hbm, srcw_hbm, dstw_hbm, zeros_hbm, agg3_hbm,
              src_v, dst_v, rows_v, zeros_v, gsem, ssem, zsem, acc_sh):
    ci = lax.axis_index("c")
    si = lax.axis_index("s")
    pltpu.sync_copy(srcw_hbm.at[si], src_v)
    pltpu.sync_copy(dstw_hbm.at[si], dst_v)
    pltpu.sync_copy(zeros_hbm, zeros_v)

    for half in range(2):
        @pl.when(ci == half)
        def _():
            base = half * (NCH // 2)
            for b in range(NBUF):
                pltpu.async_copy(h3_hbm.at[base].at[src_v.at[b]],
                                 rows_v.at[b], gsem.at[b])
            for k in range(NCH // 2):
                c = base + k

                @pl.loop(0, STRIPE, step=64)
                def _(z):
                    pltpu.async_copy(
                        zeros_v, acc_sh.at[pl.ds(si * STRIPE + z, 64)],
                        zsem)

                @pl.loop(0, STRIPE, step=64)
                def _(z):
                    pltpu.make_async_copy(
                        zeros_v, acc_sh.at[pl.ds(si * STRIPE + z, 64)],
                        zsem).wait()

                plsc.subcore_barrier()

                @pl.loop(0, NWIN, step=NBUF)
                def _(j0):
                    for b in range(NBUF):
                        j = j0 + b
                        pltpu.make_async_copy(
                            h3_hbm.at[c].at[src_v.at[j]], rows_v.at[b],
                            gsem.at[b]).wait()
                        pltpu.async_copy(rows_v.at[b],
                                         acc_sh.at[dst_v.at[j]],
                                         ssem.at[b], add=True)
                        pltpu.make_async_copy(
                            rows_v.at[b], acc_sh.at[dst_v.at[j]],
                            ssem.at[b]).wait()

                        @pl.when(j0 + NBUF < NWIN)
                        def _():
                            pltpu.async_copy(
                                h3_hbm.at[c].at[src_v.at[j + NBUF]],
                                rows_v.at[b], gsem.at[b])

                if k + 1 < NCH // 2:
                    for b in range(NBUF):
                        pltpu.async_copy(h3_hbm.at[c + 1].at[src_v.at[b]],
                                         rows_v.at[b], gsem.at[b])
                plsc.subcore_barrier()
                pltpu.sync_copy(acc_sh.at[pl.ds(si * STRIPE, STRIPE)],
                                agg3_hbm.at[c].at[pl.ds(si * STRIPE, STRIPE)])
                plsc.subcore_barrier()


def _sc_aggregate(h3, srcw, dstw, zeros_chunk):
    kern = pl.kernel(
        _agg_body,
        out_type=jax.ShapeDtypeStruct((NCH, R, C), jnp.float32),
        mesh=_mesh,
        scratch_types=[
            pltpu.VMEM((NWIN, W), jnp.int32),
            pltpu.VMEM((NWIN, W), jnp.int32),
            pltpu.VMEM((NBUF, W, C), jnp.float32),
            pltpu.VMEM((64, C), jnp.float32),
            pltpu.SemaphoreType.DMA((NBUF,)),
            pltpu.SemaphoreType.DMA((NBUF,)),
            pltpu.SemaphoreType.DMA,
            pltpu.VMEM_SHARED((R, C), jnp.float32),
        ],
    )
    return kern(h3, srcw, dstw, zeros_chunk)


BR = 1024
BC = 512


KC = 256
NKC = D // KC


def _p1_body(a3_ref, w_ref, b_ref, o_ref):
    k = pl.program_id(1)
    a3 = a3_ref[...]
    a = jnp.concatenate([a3[cc] for cc in range(KC // C)], axis=1)
    part = lax.dot_general(a.astype(jnp.bfloat16), w_ref[...],
                           (((1,), (1,)), ((), ())),
                           preferred_element_type=jnp.float32)

    @pl.when(k == 0)
    def _():
        o_ref[...] = part + b_ref[...]

    @pl.when(k > 0)
    def _():
        o_ref[...] += part


def _tc_p1(h3, w_bf16, b_row):
    return pl.pallas_call(
        _p1_body,
        grid=(R // BR, NKC),
        in_specs=[
            pl.BlockSpec((KC // C, BR, C), lambda i, k: (k, i, 0)),
            pl.BlockSpec((D, KC), lambda i, k: (0, k)),
            pl.BlockSpec((1, D), lambda i, k: (0, 0)),
        ],
        out_specs=pl.BlockSpec((BR, D), lambda i, k: (i, 0)),
        out_shape=jax.ShapeDtypeStruct((R, D), jnp.float32),
    )(h3, w_bf16, b_row)


def _p2_body(tmp_ref, a3_ref, deg_ref, w_ref, o_ref, acc_ref, *, relu,
             emit_std):
    k = pl.program_id(1)
    scale = 1.0 / jnp.maximum(deg_ref[...], 1.0)
    a3 = a3_ref[...]
    a = jnp.concatenate([a3[cc] for cc in range(KC // C)], axis=1)
    a = (a * scale).astype(jnp.bfloat16)
    part = lax.dot_general(a, w_ref[...], (((1,), (1,)), ((), ())),
                           preferred_element_type=jnp.float32)

    @pl.when(k == 0)
    def _():
        acc_ref[...] = tmp_ref[...] + part

    @pl.when(k > 0)
    def _():
        acc_ref[...] += part

    @pl.when(k == NKC - 1)
    def _():
        out = acc_ref[...]
        if relu:
            out = jnp.maximum(out, 0.0)
        if emit_std:
            o_ref[...] = out
        else:
            for cc in range(NCH):
                o_ref[cc] = out[:, cc * C:(cc + 1) * C]


def _tc_p2(tmp, agg3, deg_col, w_bf16, relu, emit_std):
    if emit_std:
        out_spec = pl.BlockSpec((BR, D), lambda i, k: (i, 0))
        out_shape = jax.ShapeDtypeStruct((R, D), jnp.float32)
    else:
        out_spec = pl.BlockSpec((NCH, BR, C), lambda i, k: (0, i, 0))
        out_shape = jax.ShapeDtypeStruct((NCH, R, C), jnp.float32)
    return pl.pallas_call(
        functools.partial(_p2_body, relu=relu, emit_std=emit_std),
        grid=(R // BR, NKC),
        in_specs=[
            pl.BlockSpec((BR, D), lambda i, k: (i, 0)),
            pl.BlockSpec((KC // C, BR, C), lambda i, k: (k, i, 0)),
            pl.BlockSpec((BR, 1), lambda i, k: (i, 0)),
            pl.BlockSpec((D, KC), lambda i, k: (0, k)),
        ],
        out_specs=out_spec,
        out_shape=out_shape,
        scratch_shapes=[pltpu.VMEM((BR, D), jnp.float32)],
    )(tmp, agg3, deg_col, w_bf16)


def _chunk_body(x_ref, o_ref):
    o_ref[...] = x_ref[...][None]


def _tc_chunk(h):
    return pl.pallas_call(
        _chunk_body,
        grid=(NCH, R // BR),
        in_specs=[pl.BlockSpec((BR, C), lambda c, i: (i, c))],
        out_specs=pl.BlockSpec((1, BR, C), lambda c, i: (c, i, 0)),
        out_shape=jax.ShapeDtypeStruct((NCH, R, C), jnp.float32),
    )(h)


def kernel(x, edge_index, W_l_0, W_r_0, b_0, W_l_1, W_r_1, b_1,
           W_l_2, W_r_2, b_2, W_l_3, W_r_3, b_3):
    src = edge_index[0]
    dst = edge_index[1]
    npad = EP - E
    pad_src = jnp.zeros((npad,), jnp.int32)
    pad_dst = N + (jnp.arange(npad, dtype=jnp.int32) % (R - N))
    srcw = jnp.concatenate([src, pad_src]).reshape(16, NWIN, W)
    dstw = jnp.concatenate([dst, pad_dst]).reshape(16, NWIN, W)

    zeros_stripe = jnp.zeros((STRIPE, C), jnp.float32)
    zeros64 = jnp.zeros((64, C), jnp.float32)
    ones_chunk = jnp.ones((W, C), jnp.float32)

    Wl = [W_l_0, W_l_1, W_l_2, W_l_3]
    Wr = [W_r_0, W_r_1, W_r_2, W_r_3]
    bs = [b_0, b_1, b_2, b_3]
    Wlb = [w.astype(jnp.bfloat16) for w in Wl]
    Wrb = [w.astype(jnp.bfloat16) for w in Wr]
    brow = [b.reshape(1, D) for b in bs]

    degw = _sc_degree(dstw, ones_chunk, zeros_stripe)
    deg_col = degw[:, 0:1]

    h3 = _tc_chunk(jnp.pad(x, ((0, R - N), (0, 0))))
    for l in range(4):
        agg3 = _sc_aggregate(h3, srcw, dstw, zeros64)
        tmp = _tc_p1(h3, Wrb[l], brow[l])
        out = _tc_p2(tmp, agg3, deg_col, Wlb[l],
                     relu=(l % 2 == 0), emit_std=(l == 3))
        h3 = out
    return out[:N]
